# baseline (speedup 1.0000x reference)
import functools

import jax
import jax.numpy as jnp
from jax import lax
from jax.experimental import pallas as pl
from jax.experimental.pallas import tpu as pltpu
from jax.experimental.pallas import tpu_sc as plsc

_NC = 2
_NS = 16
_NW = _NC * _NS
_B = 128


def _sc_mesh():
    return plsc.VectorSubcoreMesh(core_axis_name="c", subcore_axis_name="s",
                                  num_cores=_NC, num_subcores=_NS)


def _sc_degree(col2d, acc_rows):
    kr_w = col2d.shape[0] // _NW
    L = 16
    seg = acc_rows // _NS

    @functools.partial(
        pl.kernel,
        out_type=jax.ShapeDtypeStruct((_NC, acc_rows, _B), jnp.float32),
        mesh=_sc_mesh(),
        scratch_types=[
            pltpu.VMEM((kr_w, _B), jnp.int32),
            pltpu.VMEM((_B, _B), jnp.float32),
            pltpu.VMEM_SHARED((acc_rows, _B), jnp.float32),
        ],
    )
    def deg_kernel(col_hbm, out_hbm, col_v, ones_v, acc):
        cid = lax.axis_index("c")
        sid = lax.axis_index("s")
        wid = sid * _NC + cid
        zvec = jnp.zeros((L,), jnp.float32)
        for r in range(_B):
            for k in range(_B // L):
                ones_v[r, pl.ds(k * L, L)] = zvec
        for s in range(seg // _B):
            pltpu.sync_copy(ones_v, acc.at[pl.ds(sid * seg + s * _B, _B)])
        ones = jnp.ones((L,), jnp.float32)
        for r in range(_B):
            for k in range(_B // L):
                ones_v[r, pl.ds(k * L, L)] = ones
        pltpu.sync_copy(col_hbm.at[pl.ds(wid * kr_w, kr_w)], col_v)
        plsc.subcore_barrier()

        def body(j, carry):
            pltpu.sync_copy(ones_v, acc.at[col_v.at[j]], add=True)
            return carry

        lax.fori_loop(0, kr_w, body, 0)
        plsc.subcore_barrier()
        pltpu.sync_copy(acc.at[pl.ds(sid * seg, seg)],
                        out_hbm.at[cid, pl.ds(sid * seg, seg)])

    return deg_kernel(col2d)


_KR0 = 120
_KR1 = 40
_SEG = 40


def _sc_scatter_sum(g, row2d, col2d, zeros_blk):
    n_nodes, d = g.shape
    zr = zeros_blk.shape[0]
    acc_rows = zr * _NS
    assert row2d.shape[0] == _NS * (_KR0 + _KR1)

    @functools.partial(
        pl.kernel,
        out_type=jax.ShapeDtypeStruct((_NC, acc_rows, d), jnp.float32),
        mesh=_sc_mesh(),
        scratch_types=[
            pltpu.VMEM((_SEG, _B), jnp.int32),
            pltpu.VMEM((_SEG, _B), jnp.int32),
            pltpu.VMEM((_B, d), jnp.float32),
            pltpu.VMEM((_B, d), jnp.float32),
            pltpu.VMEM_SHARED((zr * _NS, d), jnp.float32),
            pltpu.SemaphoreType.DMA,
            pltpu.SemaphoreType.DMA,
        ],
    )
    def edge_kernel(g_hbm, row_hbm, col_hbm, zeros_hbm, out_hbm,
                    row_v, col_v, rows0, rows1, acc, gs0, gs1):
        cid = lax.axis_index("c")
        sid = lax.axis_index("s")
        pltpu.sync_copy(zeros_hbm, acc.at[pl.ds(sid * zr, zr)])
        plsc.subcore_barrier()

        def body(p, carry):
            j0 = 2 * p
            d0 = pltpu.async_copy(g_hbm.at[row_v.at[j0]], rows0, gs0)

            @pl.when(p >= 1)
            def _():
                pltpu.sync_copy(rows1, acc.at[col_v.at[j0 - 1]], add=True)

            d0.wait()
            d1 = pltpu.async_copy(g_hbm.at[row_v.at[j0 + 1]], rows1, gs1)
            pltpu.sync_copy(rows0, acc.at[col_v.at[j0]], add=True)
            d1.wait()
            return carry

        def run_segment(base):
            pltpu.sync_copy(row_hbm.at[pl.ds(base, _SEG)], row_v)
            pltpu.sync_copy(col_hbm.at[pl.ds(base, _SEG)], col_v)
            lax.fori_loop(0, _SEG // 2, body, 0)
            pltpu.sync_copy(rows1, acc.at[col_v.at[_SEG - 1]], add=True)

        @pl.when(cid == 0)
        def _():
            for s in range(_KR0 // _SEG):
                run_segment(sid * _KR0 + s * _SEG)

        @pl.when(cid == 1)
        def _():
            for s in range(_KR1 // _SEG):
                run_segment(_NS * _KR0 + sid * _KR1 + s * _SEG)

        plsc.subcore_barrier()
        pltpu.sync_copy(acc.at[pl.ds(sid * zr, zr)],
                        out_hbm.at[cid, pl.ds(sid * zr, zr)])

    return edge_kernel(g, row2d, col2d, zeros_blk)


_GR0 = 16
_GR1 = 8


def _sc_gather_rows(z, idx2d):
    n_nodes, d = z.shape
    kr = idx2d.shape[0]
    assert kr == _NS * (_GR0 + _GR1)

    @functools.partial(
        pl.kernel,
        out_type=jax.ShapeDtypeStruct((kr * _B, d), jnp.float32),
        mesh=_sc_mesh(),
        scratch_types=[
            pltpu.VMEM((_GR0, _B), jnp.int32),
            pltpu.VMEM((_B, d), jnp.float32),
            pltpu.VMEM((_B, d), jnp.float32),
            pltpu.SemaphoreType.DMA,
            pltpu.SemaphoreType.DMA,
        ],
    )
    def gather_kernel(z_hbm, idx_hbm, out_hbm, idx_v, rows0, rows1, gs0, gs1):
        cid = lax.axis_index("c")
        sid = lax.axis_index("s")

        def run_rows(base, nrows):
            pltpu.sync_copy(idx_hbm.at[pl.ds(base, nrows)],
                            idx_v.at[pl.ds(0, nrows)])

            def body(p, carry):
                j0 = 2 * p
                d0 = pltpu.async_copy(z_hbm.at[idx_v.at[j0]], rows0, gs0)

                @pl.when(p >= 1)
                def _():
                    pltpu.sync_copy(
                        rows1, out_hbm.at[pl.ds((base + j0 - 1) * _B, _B)])

                d0.wait()
                d1 = pltpu.async_copy(z_hbm.at[idx_v.at[j0 + 1]], rows1, gs1)
                pltpu.sync_copy(
                    rows0, out_hbm.at[pl.ds((base + j0) * _B, _B)])
                d1.wait()
                return carry

            lax.fori_loop(0, nrows // 2, body, 0)
            pltpu.sync_copy(
                rows1, out_hbm.at[pl.ds((base + nrows - 1) * _B, _B)])

        @pl.when(cid == 0)
        def _():
            run_rows(sid * _GR0, _GR0)

        @pl.when(cid == 1)
        def _():
            run_rows(_NS * _GR0 + sid * _GR1, _GR1)

    return gather_kernel(z, idx2d)


def _tc_encode_in(x, W, deg):
    n, d = x.shape
    R = 1000

    def body(x_ref, w_ref, deg_ref, o_ref):
        dis = lax.rsqrt(deg_ref[...])
        h = jnp.dot(x_ref[...], w_ref[...], preferred_element_type=jnp.float32)
        o_ref[...] = h * dis

    return pl.pallas_call(
        body,
        grid=(n // R,),
        in_specs=[pl.BlockSpec((R, d), lambda i: (i, 0)),
                  pl.BlockSpec((d, W.shape[1]), lambda i: (0, 0)),
                  pl.BlockSpec((R, 1), lambda i: (i, 0))],
        out_specs=pl.BlockSpec((R, W.shape[1]), lambda i: (i, 0)),
        out_shape=jax.ShapeDtypeStruct((n, W.shape[1]), jnp.float32),
    )(x, W, deg)


def _tc_layer_mid(acc, g, deg, b, W):
    n, d = g.shape
    R = 1000

    def body(acc_ref, g_ref, deg_ref, b_ref, w_ref, o_ref):
        dis = lax.rsqrt(deg_ref[...])
        z = (acc_ref[0] + acc_ref[1] + g_ref[...]) * dis + b_ref[...]
        z = jnp.maximum(z, 0.0)
        h = jnp.dot(z, w_ref[...], preferred_element_type=jnp.float32)
        o_ref[...] = h * dis

    return pl.pallas_call(
        body,
        grid=(n // R,),
        in_specs=[pl.BlockSpec((_NC, R, d), lambda i: (0, i, 0)),
                  pl.BlockSpec((R, d), lambda i: (i, 0)),
                  pl.BlockSpec((R, 1), lambda i: (i, 0)),
                  pl.BlockSpec((1, d), lambda i: (0, 0)),
                  pl.BlockSpec((d, W.shape[1]), lambda i: (0, 0))],
        out_specs=pl.BlockSpec((R, W.shape[1]), lambda i: (i, 0)),
        out_shape=jax.ShapeDtypeStruct((n, W.shape[1]), jnp.float32),
    )(acc, g, deg, b, W)


def _tc_layer_out(acc, g, deg, b):
    n, d = g.shape
    R = 1000

    def body(acc_ref, g_ref, deg_ref, b_ref, o_ref):
        dis = lax.rsqrt(deg_ref[...])
        o_ref[...] = (acc_ref[0] + acc_ref[1] + g_ref[...]) * dis + b_ref[...]

    return pl.pallas_call(
        body,
        grid=(n // R,),
        in_specs=[pl.BlockSpec((_NC, R, d), lambda i: (0, i, 0)),
                  pl.BlockSpec((R, d), lambda i: (i, 0)),
                  pl.BlockSpec((R, 1), lambda i: (i, 0)),
                  pl.BlockSpec((1, d), lambda i: (0, 0))],
        out_specs=pl.BlockSpec((R, d), lambda i: (i, 0)),
        out_shape=jax.ShapeDtypeStruct((n, d), jnp.float32),
    )(acc, g, deg, b)


def _tc_decoder(za, zb, A, Bm, b1, w2, b2):
    n, d = za.shape
    R = 1024

    def body(za_ref, zb_ref, a_ref, bm_ref, b1_ref, w2_ref, b2_ref, o_ref):
        zaa = za_ref[...]
        zbb = zb_ref[...]
        am = a_ref[...]
        bm = bm_ref[...]
        s1 = (jnp.dot(zaa, am, preferred_element_type=jnp.float32)
              + jnp.dot(zbb, bm, preferred_element_type=jnp.float32)
              + b1_ref[...])
        s2 = (jnp.dot(zbb, am, preferred_element_type=jnp.float32)
              + jnp.dot(zaa, bm, preferred_element_type=jnp.float32)
              + b1_ref[...])
        w2row = w2_ref[...]
        o1 = jnp.sum(jnp.maximum(s1, 0.0) * w2row, axis=1)
        o2 = jnp.sum(jnp.maximum(s2, 0.0) * w2row, axis=1)
        o = 0.5 * (o1 + o2) + b2_ref[0, 0]
        o_ref[...] = jnp.concatenate([(-o)[None, :], o[None, :]], axis=0)

    return pl.pallas_call(
        body,
        grid=(n // R,),
        in_specs=[pl.BlockSpec((R, d), lambda i: (i, 0)),
                  pl.BlockSpec((R, d), lambda i: (i, 0)),
                  pl.BlockSpec((d, d), lambda i: (0, 0)),
                  pl.BlockSpec((d, d), lambda i: (0, 0)),
                  pl.BlockSpec((1, d), lambda i: (0, 0)),
                  pl.BlockSpec((1, d), lambda i: (0, 0)),
                  pl.BlockSpec((1, 1), lambda i: (0, 0))],
        out_specs=pl.BlockSpec((2, R), lambda i: (0, i)),
        out_shape=jax.ShapeDtypeStruct((2, n), jnp.float32),
    )(za, zb, A, Bm, b1, w2, b2)


def kernel(x, edge_index, edge_label_index, Wc1, bc1, Wc2, bc2, Wd1, bd1, Wd2, bd2):
    n, d = x.shape
    e = edge_index.shape[1]
    el = edge_label_index.shape[1]
    assert n % _NS == 0 and d % 128 == 0

    acc_rows = -(-(n + 1) // (_NS * 128)) * (_NS * 128)
    ep = _NS * (_KR0 + _KR1) * _B
    assert e <= ep
    row = edge_index[0]
    col = edge_index[1]
    if ep != e:
        dump = n + jnp.arange(ep - e, dtype=jnp.int32) % (acc_rows - n)
        row = jnp.concatenate([row, jnp.zeros((ep - e,), jnp.int32)])
        col = jnp.concatenate([col, dump])
    row2d = row.reshape(-1, _B)
    col2d = col.reshape(-1, _B)
    zr = acc_rows // _NS
    zeros_d = jnp.zeros((zr, d), jnp.float32)

    deg2 = _sc_degree(col2d, acc_rows)
    deg = (deg2[0, :n, 0] + deg2[1, :n, 0] + 1.0).reshape(n, 1)

    g1 = _tc_encode_in(x, Wc1, deg)
    acc1 = _sc_scatter_sum(g1, row2d, col2d, zeros_d)
    g2 = _tc_layer_mid(acc1, g1, deg, bc1.reshape(1, -1), Wc2)
    acc2 = _sc_scatter_sum(g2, row2d, col2d, zeros_d)
    z = _tc_layer_out(acc2, g2, deg, bc2.reshape(1, -1))

    half = _NS * (_GR0 + _GR1) * _B // 2
    assert el <= half
    pad = jnp.zeros((half - el,), jnp.int32)
    eli = jnp.concatenate(
        [edge_label_index[0], pad, edge_label_index[1], pad]).reshape(-1, _B)
    zcat = _sc_gather_rows(z, eli)
    za = zcat[:half]
    zb = zcat[half:]

    out = _tc_decoder(za, zb, Wd1[:d], Wd1[d:], bd1.reshape(1, -1),
                      Wd2.reshape(1, -1), bd2.reshape(1, 1))
    return out[:, :el]


# speedup vs baseline: 1.2300x; 1.2300x over previous
import functools

import jax
import jax.numpy as jnp
from jax import lax
from jax.experimental import pallas as pl
from jax.experimental.pallas import tpu as pltpu
from jax.experimental.pallas import tpu_sc as plsc

_NC = 2
_NS = 16
_NW = _NC * _NS
_B = 128


def _sc_mesh():
    return plsc.VectorSubcoreMesh(core_axis_name="c", subcore_axis_name="s",
                                  num_cores=_NC, num_subcores=_NS)


def _sc_degree(col2d, acc_rows):
    kr_w = col2d.shape[0] // _NW
    L = 16
    seg = acc_rows // _NS

    @functools.partial(
        pl.kernel,
        out_type=jax.ShapeDtypeStruct((_NC, acc_rows, _B), jnp.float32),
        mesh=_sc_mesh(),
        scratch_types=[
            pltpu.VMEM((kr_w, _B), jnp.int32),
            pltpu.VMEM((_B, _B), jnp.float32),
            pltpu.VMEM_SHARED((acc_rows, _B), jnp.float32),
        ],
    )
    def deg_kernel(col_hbm, out_hbm, col_v, ones_v, acc):
        cid = lax.axis_index("c")
        sid = lax.axis_index("s")
        wid = sid * _NC + cid
        zvec = jnp.zeros((L,), jnp.float32)
        for r in range(_B):
            for k in range(_B // L):
                ones_v[r, pl.ds(k * L, L)] = zvec
        for s in range(seg // _B):
            pltpu.sync_copy(ones_v, acc.at[pl.ds(sid * seg + s * _B, _B)])
        ones = jnp.ones((L,), jnp.float32)
        for r in range(_B):
            for k in range(_B // L):
                ones_v[r, pl.ds(k * L, L)] = ones
        pltpu.sync_copy(col_hbm.at[pl.ds(wid * kr_w, kr_w)], col_v)
        plsc.subcore_barrier()

        def body(j, carry):
            pltpu.sync_copy(ones_v, acc.at[col_v.at[j]], add=True)
            return carry

        lax.fori_loop(0, kr_w, body, 0)
        plsc.subcore_barrier()
        pltpu.sync_copy(acc.at[pl.ds(sid * seg, seg)],
                        out_hbm.at[cid, pl.ds(sid * seg, seg)])

    return deg_kernel(col2d)


_KR0 = 80
_KR1 = 80
_SEG = 40


def _sc_scatter_sum(g, row2d, col2d, zeros_blk):
    n_nodes, d = g.shape
    zr = zeros_blk.shape[0]
    acc_rows = zr * _NS
    assert row2d.shape[0] == _NS * (_KR0 + _KR1)

    @functools.partial(
        pl.kernel,
        out_type=jax.ShapeDtypeStruct((_NC, acc_rows, d), jnp.float32),
        mesh=_sc_mesh(),
        scratch_types=[
            pltpu.VMEM((_SEG, _B), jnp.int32),
            pltpu.VMEM((_SEG, _B), jnp.int32),
            pltpu.VMEM((_B, d), jnp.float32),
            pltpu.VMEM((_B, d), jnp.float32),
            pltpu.VMEM_SHARED((zr * _NS, d), jnp.float32),
            pltpu.SemaphoreType.DMA,
            pltpu.SemaphoreType.DMA,
        ],
    )
    def edge_kernel(g_hbm, row_hbm, col_hbm, zeros_hbm, out_hbm,
                    row_v, col_v, rows0, rows1, acc, gs0, gs1):
        cid = lax.axis_index("c")
        sid = lax.axis_index("s")
        pltpu.sync_copy(zeros_hbm, acc.at[pl.ds(sid * zr, zr)])
        plsc.subcore_barrier()

        def body(p, carry):
            j0 = 2 * p
            d0 = pltpu.async_copy(g_hbm.at[row_v.at[j0]], rows0, gs0)

            @pl.when(p >= 1)
            def _():
                pltpu.sync_copy(rows1, acc.at[col_v.at[j0 - 1]], add=True)

            d0.wait()
            d1 = pltpu.async_copy(g_hbm.at[row_v.at[j0 + 1]], rows1, gs1)
            pltpu.sync_copy(rows0, acc.at[col_v.at[j0]], add=True)
            d1.wait()
            return carry

        def run_segment(base):
            pltpu.sync_copy(row_hbm.at[pl.ds(base, _SEG)], row_v)
            pltpu.sync_copy(col_hbm.at[pl.ds(base, _SEG)], col_v)
            lax.fori_loop(0, _SEG // 2, body, 0)
            pltpu.sync_copy(rows1, acc.at[col_v.at[_SEG - 1]], add=True)

        @pl.when(cid == 0)
        def _():
            for s in range(_KR0 // _SEG):
                run_segment(sid * _KR0 + s * _SEG)

        @pl.when(cid == 1)
        def _():
            for s in range(_KR1 // _SEG):
                run_segment(_NS * _KR0 + sid * _KR1 + s * _SEG)

        plsc.subcore_barrier()
        pltpu.sync_copy(acc.at[pl.ds(sid * zr, zr)],
                        out_hbm.at[cid, pl.ds(sid * zr, zr)])

    return edge_kernel(g, row2d, col2d, zeros_blk)


_GR0 = 16


def _sc_gather_rows(z, idx2d):
    n_nodes, d = z.shape
    kr = idx2d.shape[0]
    n_active = kr // _GR0
    assert kr % _GR0 == 0 and n_active <= _NW

    @functools.partial(
        pl.kernel,
        out_type=jax.ShapeDtypeStruct((kr * _B, d), jnp.float32),
        mesh=_sc_mesh(),
        scratch_types=[
            pltpu.VMEM((_GR0, _B), jnp.int32),
            pltpu.VMEM((_B, d), jnp.float32),
            pltpu.VMEM((_B, d), jnp.float32),
            pltpu.SemaphoreType.DMA,
            pltpu.SemaphoreType.DMA,
        ],
    )
    def gather_kernel(z_hbm, idx_hbm, out_hbm, idx_v, rows0, rows1, gs0, gs1):
        cid = lax.axis_index("c")
        sid = lax.axis_index("s")

        def run_rows(base, nrows):
            pltpu.sync_copy(idx_hbm.at[pl.ds(base, nrows)],
                            idx_v.at[pl.ds(0, nrows)])

            def body(p, carry):
                j0 = 2 * p
                d0 = pltpu.async_copy(z_hbm.at[idx_v.at[j0]], rows0, gs0)

                @pl.when(p >= 1)
                def _():
                    pltpu.sync_copy(
                        rows1, out_hbm.at[pl.ds((base + j0 - 1) * _B, _B)])

                d0.wait()
                d1 = pltpu.async_copy(z_hbm.at[idx_v.at[j0 + 1]], rows1, gs1)
                pltpu.sync_copy(
                    rows0, out_hbm.at[pl.ds((base + j0) * _B, _B)])
                d1.wait()
                return carry

            lax.fori_loop(0, nrows // 2, body, 0)
            pltpu.sync_copy(
                rows1, out_hbm.at[pl.ds((base + nrows - 1) * _B, _B)])

        wid = sid * _NC + cid

        @pl.when(wid < n_active)
        def _():
            run_rows(wid * _GR0, _GR0)

    return gather_kernel(z, idx2d)


def _tc_encode_in(x, W, deg):
    n, d = x.shape
    R = 1000

    def body(x_ref, w_ref, deg_ref, o_ref):
        dis = lax.rsqrt(deg_ref[...])
        h = jnp.dot(x_ref[...], w_ref[...], preferred_element_type=jnp.float32)
        o_ref[...] = h * dis

    return pl.pallas_call(
        body,
        grid=(n // R,),
        in_specs=[pl.BlockSpec((R, d), lambda i: (i, 0)),
                  pl.BlockSpec((d, W.shape[1]), lambda i: (0, 0)),
                  pl.BlockSpec((R, 1), lambda i: (i, 0))],
        out_specs=pl.BlockSpec((R, W.shape[1]), lambda i: (i, 0)),
        out_shape=jax.ShapeDtypeStruct((n, W.shape[1]), jnp.float32),
    )(x, W, deg)


def _tc_layer_mid(acc, g, deg, b, W):
    n, d = g.shape
    R = 1000

    def body(acc_ref, g_ref, deg_ref, b_ref, w_ref, o_ref):
        dis = lax.rsqrt(deg_ref[...])
        z = (acc_ref[0] + acc_ref[1] + g_ref[...]) * dis + b_ref[...]
        z = jnp.maximum(z, 0.0)
        h = jnp.dot(z, w_ref[...], preferred_element_type=jnp.float32)
        o_ref[...] = h * dis

    return pl.pallas_call(
        body,
        grid=(n // R,),
        in_specs=[pl.BlockSpec((_NC, R, d), lambda i: (0, i, 0)),
                  pl.BlockSpec((R, d), lambda i: (i, 0)),
                  pl.BlockSpec((R, 1), lambda i: (i, 0)),
                  pl.BlockSpec((1, d), lambda i: (0, 0)),
                  pl.BlockSpec((d, W.shape[1]), lambda i: (0, 0))],
        out_specs=pl.BlockSpec((R, W.shape[1]), lambda i: (i, 0)),
        out_shape=jax.ShapeDtypeStruct((n, W.shape[1]), jnp.float32),
    )(acc, g, deg, b, W)


def _tc_layer_out(acc, g, deg, b):
    n, d = g.shape
    R = 1000

    def body(acc_ref, g_ref, deg_ref, b_ref, o_ref):
        dis = lax.rsqrt(deg_ref[...])
        o_ref[...] = (acc_ref[0] + acc_ref[1] + g_ref[...]) * dis + b_ref[...]

    return pl.pallas_call(
        body,
        grid=(n // R,),
        in_specs=[pl.BlockSpec((_NC, R, d), lambda i: (0, i, 0)),
                  pl.BlockSpec((R, d), lambda i: (i, 0)),
                  pl.BlockSpec((R, 1), lambda i: (i, 0)),
                  pl.BlockSpec((1, d), lambda i: (0, 0))],
        out_specs=pl.BlockSpec((R, d), lambda i: (i, 0)),
        out_shape=jax.ShapeDtypeStruct((n, d), jnp.float32),
    )(acc, g, deg, b)


def _tc_decoder(za, zb, A, Bm, b1, w2, b2):
    n, d = za.shape
    R = 1024

    def body(za_ref, zb_ref, a_ref, bm_ref, b1_ref, w2_ref, b2_ref, o_ref):
        zaa = za_ref[...]
        zbb = zb_ref[...]
        am = a_ref[...]
        bm = bm_ref[...]
        s1 = (jnp.dot(zaa, am, preferred_element_type=jnp.float32)
              + jnp.dot(zbb, bm, preferred_element_type=jnp.float32)
              + b1_ref[...])
        s2 = (jnp.dot(zbb, am, preferred_element_type=jnp.float32)
              + jnp.dot(zaa, bm, preferred_element_type=jnp.float32)
              + b1_ref[...])
        w2row = w2_ref[...]
        o1 = jnp.sum(jnp.maximum(s1, 0.0) * w2row, axis=1)
        o2 = jnp.sum(jnp.maximum(s2, 0.0) * w2row, axis=1)
        o = 0.5 * (o1 + o2) + b2_ref[0, 0]
        o_ref[...] = jnp.concatenate([(-o)[None, :], o[None, :]], axis=0)

    return pl.pallas_call(
        body,
        grid=(n // R,),
        in_specs=[pl.BlockSpec((R, d), lambda i: (i, 0)),
                  pl.BlockSpec((R, d), lambda i: (i, 0)),
                  pl.BlockSpec((d, d), lambda i: (0, 0)),
                  pl.BlockSpec((d, d), lambda i: (0, 0)),
                  pl.BlockSpec((1, d), lambda i: (0, 0)),
                  pl.BlockSpec((1, d), lambda i: (0, 0)),
                  pl.BlockSpec((1, 1), lambda i: (0, 0))],
        out_specs=pl.BlockSpec((2, R), lambda i: (0, i)),
        out_shape=jax.ShapeDtypeStruct((2, n), jnp.float32),
    )(za, zb, A, Bm, b1, w2, b2)


def kernel(x, edge_index, edge_label_index, Wc1, bc1, Wc2, bc2, Wd1, bd1, Wd2, bd2):
    n, d = x.shape
    e = edge_index.shape[1]
    el = edge_label_index.shape[1]
    assert n % _NS == 0 and d % 128 == 0

    acc_rows = -(-(n + 1) // (_NS * 128)) * (_NS * 128)
    ep = _NS * (_KR0 + _KR1) * _B
    assert e <= ep
    row = edge_index[0]
    col = edge_index[1]
    if ep != e:
        dump = n + jnp.arange(ep - e, dtype=jnp.int32) % (acc_rows - n)
        row = jnp.concatenate([row, jnp.zeros((ep - e,), jnp.int32)])
        col = jnp.concatenate([col, dump])
    row2d = row.reshape(-1, _B)
    col2d = col.reshape(-1, _B)
    zr = acc_rows // _NS
    zeros_d = jnp.zeros((zr, d), jnp.float32)

    deg2 = _sc_degree(col2d, acc_rows)
    deg = (deg2[0, :n, 0] + deg2[1, :n, 0] + 1.0).reshape(n, 1)

    g1 = _tc_encode_in(x, Wc1, deg)
    acc1 = _sc_scatter_sum(g1, row2d, col2d, zeros_d)
    g2 = _tc_layer_mid(acc1, g1, deg, bc1.reshape(1, -1), Wc2)
    acc2 = _sc_scatter_sum(g2, row2d, col2d, zeros_d)
    z = _tc_layer_out(acc2, g2, deg, bc2.reshape(1, -1))

    half = -(-el // (_B * _GR0)) * (_B * _GR0)
    pad = jnp.zeros((half - el,), jnp.int32)
    eli = jnp.concatenate(
        [edge_label_index[0], pad, edge_label_index[1], pad]).reshape(-1, _B)
    zcat = _sc_gather_rows(z, eli)
    za = zcat[:half]
    zb = zcat[half:]

    out = _tc_decoder(za, zb, Wd1[:d], Wd1[d:], bd1.reshape(1, -1),
                      Wd2.reshape(1, -1), bd2.reshape(1, 1))
    return out[:, :el]
